# baseline (speedup 1.0000x reference)
import functools

import jax
import jax.numpy as jnp
from jax import lax
from jax.experimental import pallas as pl
from jax.experimental.pallas import tpu as pltpu
from jax.experimental.pallas import tpu_sc as plsc

D = 1024
ROWS = 16384
NW = 32
ROWS_PER_W = ROWS // NW
C = 8
NCHUNK = ROWS_PER_W // C
NPE = 8
NX = 2
NO = 4
LANES = 16

_mesh = plsc.VectorSubcoreMesh(core_axis_name="c", subcore_axis_name="s")


@functools.partial(
    pl.kernel,
    mesh=_mesh,
    out_type=jax.ShapeDtypeStruct((ROWS, D), jnp.float32),
    scratch_types=[
        pltpu.VMEM((NCHUNK, C), jnp.int32),
        pltpu.VMEM((NX, C, D), jnp.float32),
        pltpu.VMEM((NPE, C, D), jnp.float32),
        pltpu.VMEM((NO, C, D), jnp.float32),
    ] + [pltpu.SemaphoreType.DMA] * (NX + NPE + NO),
)
def _sc_kernel(x_hbm, idx_hbm, pe_hbm, out_hbm, idx_v, xbuf, pebuf, obuf, *sems):
    sem_x = sems[:NX]
    sem_pe = sems[NX:NX + NPE]
    sem_out = sems[NX + NPE:]
    wid = lax.axis_index("s") * 2 + lax.axis_index("c")
    base = wid * ROWS_PER_W
    pltpu.sync_copy(idx_hbm.at[wid], idx_v)

    def start_pe(c, b):
        pltpu.async_copy(pe_hbm.at[idx_v.at[c]], pebuf.at[b], sem_pe[b])

    def start_x(c, sx):
        pltpu.async_copy(x_hbm.at[pl.ds(base + c * C, C)], xbuf.at[sx],
                         sem_x[sx])

    def wait_pe(b):
        pltpu.make_async_copy(x_hbm.at[pl.ds(0, C)], pebuf.at[b], sem_pe[b]).wait()

    def wait_x(sx):
        pltpu.make_async_copy(x_hbm.at[pl.ds(0, C)], xbuf.at[sx], sem_x[sx]).wait()

    def wait_out(so):
        pltpu.make_async_copy(x_hbm.at[pl.ds(0, C)], obuf.at[so], sem_out[so]).wait()

    for b in range(NPE):
        start_pe(b, b)
    for sx in range(NX):
        start_x(sx, sx)

    @pl.loop(0, NCHUNK, step=NPE)
    def _outer(o):
        for b in range(NPE):
            c = o + b
            sx = b % NX
            so = b % NO
            wait_pe(b)
            wait_x(sx)

            if b >= NO:
                wait_out(so)
            else:
                @pl.when(o >= NPE)
                def _():
                    wait_out(so)

            @pl.loop(0, C)
            def _row(j):
                for g in range(D // LANES):
                    sl = pl.ds(g * LANES, LANES)
                    obuf[so, j, sl] = xbuf[sx, j, sl] + pebuf[b, j, sl]

            pltpu.async_copy(obuf.at[so], out_hbm.at[pl.ds(base + c * C, C)],
                             sem_out[so])

            @pl.when(c + NX < NCHUNK)
            def _():
                start_x(c + NX, sx)

            @pl.when(c + NPE < NCHUNK)
            def _():
                start_pe(c + NPE, b)

    for so in range(NO):
        wait_out(so)


def kernel(x, pos_id_torch_pad, pe_table):
    xf = x.reshape(ROWS, D)
    idx = pos_id_torch_pad.astype(jnp.int32).reshape(NW, NCHUNK, C)
    out = _sc_kernel(xf, idx, pe_table)
    return out.reshape(x.shape)


# speedup vs baseline: 1.1495x; 1.1495x over previous
import functools

import jax
import jax.numpy as jnp
from jax import lax
from jax.experimental import pallas as pl
from jax.experimental.pallas import tpu as pltpu
from jax.experimental.pallas import tpu_sc as plsc

D = 1024
ROWS = 16384
NW = 32
ROWS_PER_W = ROWS // NW
C = 8
NCHUNK = ROWS_PER_W // C
NBUF = 4
LANES = 16

_mesh = plsc.VectorSubcoreMesh(core_axis_name="c", subcore_axis_name="s")


@functools.partial(
    pl.kernel,
    mesh=_mesh,
    out_type=jax.ShapeDtypeStruct((ROWS, D), jnp.float32),
    scratch_types=[
        pltpu.VMEM((NCHUNK, C), jnp.int32),
        pltpu.VMEM((NBUF, C, D), jnp.float32),
        pltpu.VMEM((NBUF, C, D), jnp.float32),
        pltpu.VMEM((NBUF, C, D), jnp.float32),
    ] + [pltpu.SemaphoreType.DMA] * (2 * NBUF),
)
def _sc_kernel(x_hbm, idx_hbm, pe_hbm, out_hbm, idx_v, xbuf, pebuf, obuf, *sems):
    sem_in = sems[:NBUF]
    sem_out = sems[NBUF:]
    wid = lax.axis_index("s") * 2 + lax.axis_index("c")
    base = wid * ROWS_PER_W
    pltpu.sync_copy(idx_hbm.at[wid], idx_v)

    def start_in(c, b):
        off = base + c * C
        pltpu.async_copy(x_hbm.at[pl.ds(off, C)], xbuf.at[b], sem_in[b])
        pltpu.async_copy(pe_hbm.at[idx_v.at[c]], pebuf.at[b], sem_in[b])

    def wait_in(b):
        pltpu.make_async_copy(x_hbm.at[pl.ds(0, C)], xbuf.at[b], sem_in[b]).wait()
        pltpu.make_async_copy(x_hbm.at[pl.ds(0, C)], pebuf.at[b], sem_in[b]).wait()

    def wait_out(b):
        pltpu.make_async_copy(x_hbm.at[pl.ds(0, C)], obuf.at[b], sem_out[b]).wait()

    for b in range(NBUF):
        start_in(b, b)

    @pl.loop(0, NCHUNK, step=NBUF)
    def _outer(o):
        for b in range(NBUF):
            c = o + b
            @pl.when(o >= NBUF)
            def _():
                wait_out(b)
            wait_in(b)

            @pl.loop(0, C)
            def _row(j):
                for g in range(D // LANES):
                    sl = pl.ds(g * LANES, LANES)
                    obuf[b, j, sl] = xbuf[b, j, sl] + pebuf[b, j, sl]

            pltpu.async_copy(obuf.at[b], out_hbm.at[pl.ds(base + c * C, C)],
                             sem_out[b])

            @pl.when(c + NBUF < NCHUNK)
            def _():
                start_in(c + NBUF, b)

    for b in range(NBUF):
        wait_out(b)


def kernel(x, pos_id_torch_pad, pe_table):
    xf = x.reshape(ROWS, D)
    idx = pos_id_torch_pad.astype(jnp.int32).reshape(NW, NCHUNK, C)
    out = _sc_kernel(xf, idx, pe_table)
    return out.reshape(x.shape)
